# Initial kernel scaffold; baseline (speedup 1.0000x reference)
import jax
import jax.numpy as jnp
from jax.experimental import pallas as pl


def kernel(x, edge_index, W1, b1, g1, be1, W2, b2, g2, be2, W3, b3):
    raise NotImplementedError("write your pallas kernel here")


# speedup vs baseline: 8.5787x; 8.5787x over previous
import functools

import jax
import jax.numpy as jnp
from jax import lax
from jax.experimental import pallas as pl
from jax.experimental.pallas import tpu as pltpu
from jax.experimental.pallas import tpu_sc as plsc

NC = 2
NS = 16
NW = NC * NS
CH = 128
EPS = 1e-5


def _degree_call(dst_p, zeros_nd, ones2d, np_, d, n_chunks, epw):
    @functools.partial(
        pl.kernel,
        out_type=jax.ShapeDtypeStruct((NC, np_, d), jnp.float32),
        mesh=plsc.VectorSubcoreMesh(core_axis_name="c", subcore_axis_name="s",
                                    num_cores=NC, num_subcores=NS),
        scratch_types=[
            pltpu.VMEM((CH,), jnp.int32),
            pltpu.VMEM((CH, d), jnp.float32),
            pltpu.VMEM_SHARED((np_, d), jnp.float32),
        ],
    )
    def deg_kernel(dst_hbm, z_hbm, ones_hbm, out_hbm, didx, ones_v, dacc):
        cid = lax.axis_index("c")
        sid = lax.axis_index("s")
        wid = sid * NC + cid
        rpt = np_ // NS
        rbase = sid * rpt
        pltpu.sync_copy(z_hbm.at[pl.ds(rbase, rpt)], dacc.at[pl.ds(rbase, rpt)])
        pltpu.sync_copy(ones_hbm, ones_v)
        plsc.subcore_barrier()
        ebase = wid * epw

        @pl.loop(0, n_chunks)
        def _(j):
            pltpu.sync_copy(dst_hbm.at[pl.ds(ebase + j * CH, CH)], didx)
            pltpu.sync_copy(ones_v, dacc.at[didx], add=True)

        plsc.subcore_barrier()
        pltpu.sync_copy(dacc.at[pl.ds(rbase, rpt)],
                        out_hbm.at[cid].at[pl.ds(rbase, rpt)])

    return deg_kernel(dst_p, zeros_nd, ones2d)


def _aggregate_call(hs, src_p, dst_p, np_, d, n_chunks, epw):
    @functools.partial(
        pl.kernel,
        out_type=jax.ShapeDtypeStruct((NC, np_, d), jnp.float32),
        mesh=plsc.VectorSubcoreMesh(core_axis_name="c", subcore_axis_name="s",
                                    num_cores=NC, num_subcores=NS),
        scratch_types=[
            pltpu.VMEM((CH,), jnp.int32),
            pltpu.VMEM((CH,), jnp.int32),
            pltpu.VMEM((CH, d), jnp.float32),
            pltpu.VMEM_SHARED((np_, d), jnp.float32),
            pltpu.SemaphoreType.DMA,
        ],
    )
    def agg_kernel(hs_hbm, src_hbm, dst_hbm, out_hbm, sidx, didx, rows, acc, sem):
        cid = lax.axis_index("c")
        sid = lax.axis_index("s")
        wid = sid * NC + cid
        rpt = np_ // NS
        rbase = sid * rpt
        pltpu.sync_copy(hs_hbm.at[pl.ds(rbase, rpt)], acc.at[pl.ds(rbase, rpt)])
        plsc.subcore_barrier()
        ebase = wid * epw

        @pl.loop(0, n_chunks)
        def _(j):
            off = ebase + j * CH
            pltpu.sync_copy(src_hbm.at[pl.ds(off, CH)], sidx)
            pltpu.sync_copy(dst_hbm.at[pl.ds(off, CH)], didx)
            pltpu.async_copy(hs_hbm.at[sidx], rows, sem).wait()
            pltpu.sync_copy(rows, acc.at[didx], add=True)

        plsc.subcore_barrier()
        pltpu.sync_copy(acc.at[pl.ds(rbase, rpt)],
                        out_hbm.at[cid].at[pl.ds(rbase, rpt)])

    return agg_kernel(hs, src_p, dst_p)


def _matmul_call(x, w):
    def mm_kernel(x_ref, w_ref, o_ref):
        o_ref[...] = jnp.dot(x_ref[...], w_ref[...],
                             preferred_element_type=jnp.float32)

    return pl.pallas_call(
        mm_kernel,
        out_shape=jax.ShapeDtypeStruct((x.shape[0], w.shape[1]), jnp.float32),
    )(x, w)


def _scale_first_call(h, degs, np_, d):
    def k(h_ref, dg_ref, hs_ref, dinv_ref):
        deg = dg_ref[0, :, 0:1] + dg_ref[1, :, 0:1] + 1.0
        dinv = lax.rsqrt(deg)
        dinv_ref[...] = dinv
        hs_ref[...] = h_ref[...] * dinv

    return pl.pallas_call(
        k,
        out_shape=[
            jax.ShapeDtypeStruct((np_, d), jnp.float32),
            jax.ShapeDtypeStruct((np_, 1), jnp.float32),
        ],
    )(h, degs)


def _mid_layer_call(acc, hs_prev, dinv, b, g, be, w_next, n, np_, d):
    def k(acc_ref, hsp_ref, dinv_ref, b_ref, g_ref, be_ref, w_ref, o_ref):
        dinv_v = dinv_ref[...]
        agg = acc_ref[0] + acc_ref[1] - hsp_ref[...]
        y = agg * dinv_v + b_ref[...]
        yr = y[:n, :]
        mu = jnp.mean(yr, axis=0, keepdims=True)
        cr = yr - mu
        var = jnp.mean(cr * cr, axis=0, keepdims=True)
        yn = g_ref[...] * ((y - mu) * lax.rsqrt(var + EPS)) + be_ref[...]
        r = jnp.maximum(yn, 0.0)
        o_ref[...] = jnp.dot(r, w_ref[...],
                             preferred_element_type=jnp.float32) * dinv_v

    return pl.pallas_call(
        k,
        out_shape=jax.ShapeDtypeStruct((np_, d), jnp.float32),
    )(acc, hs_prev, dinv, b, g, be, w_next)


def _final_layer_call(acc, hs_prev, dinv, b, n, d):
    def k(acc_ref, hsp_ref, dinv_ref, b_ref, o_ref):
        agg = acc_ref[0, :n, :] + acc_ref[1, :n, :] - hsp_ref[:n, :]
        o_ref[...] = agg * dinv_ref[:n, :] + b_ref[...]

    return pl.pallas_call(
        k,
        out_shape=jax.ShapeDtypeStruct((n, d), jnp.float32),
    )(acc, hs_prev, dinv, b)


def kernel(x, edge_index, W1, b1, g1, be1, W2, b2, g2, be2, W3, b3):
    n, d = x.shape
    e = edge_index.shape[1]

    np_ = -(-n // 128) * 128
    epw = -(-e // (NW * CH)) * CH
    ep = epw * NW
    n_chunks = epw // CH
    pad = ep - e
    src = edge_index[0]
    dst = edge_index[1]
    src_p = jnp.concatenate([src, jnp.full((pad,), n, jnp.int32)])
    dst_p = jnp.concatenate([dst, jnp.full((pad,), n, jnp.int32)])
    zeros_nd = jnp.zeros((np_, d), jnp.float32)
    ones2d = jnp.ones((CH, d), jnp.float32)
    x_p = jnp.concatenate([x, jnp.zeros((np_ - n, d), jnp.float32)])

    degs = _degree_call(dst_p, zeros_nd, ones2d, np_, d, n_chunks, epw)
    h1 = _matmul_call(x_p, W1)
    hs1, dinv = _scale_first_call(h1, degs, np_, d)

    b1r, g1r, be1r = b1.reshape(1, d), g1.reshape(1, d), be1.reshape(1, d)
    b2r, g2r, be2r = b2.reshape(1, d), g2.reshape(1, d), be2.reshape(1, d)
    b3r = b3.reshape(1, d)

    acc1 = _aggregate_call(hs1, src_p, dst_p, np_, d, n_chunks, epw)
    hs2 = _mid_layer_call(acc1, hs1, dinv, b1r, g1r, be1r, W2, n, np_, d)
    acc2 = _aggregate_call(hs2, src_p, dst_p, np_, d, n_chunks, epw)
    hs3 = _mid_layer_call(acc2, hs2, dinv, b2r, g2r, be2r, W3, n, np_, d)
    acc3 = _aggregate_call(hs3, src_p, dst_p, np_, d, n_chunks, epw)
    return _final_layer_call(acc3, hs3, dinv, b3r, n, d)
